# baseline (speedup 1.0000x reference)
import functools

import jax
import jax.numpy as jnp
from jax import lax
from jax.experimental import pallas as pl
from jax.experimental.pallas import tpu as pltpu
from jax.experimental.pallas import tpu_sc as plsc

_B, _N, _K = 4, 5, 3
_C, _H, _W = 256, 96, 96
_CHW = _C * _H * _W
_CL = 3072
_SPLIT = _CHW // _CL
_ROWS_OUT = _B * _K
_TOTAL_CHUNKS = _ROWS_OUT * _SPLIT
_NC, _NS = 2, 16
_NW = _NC * _NS
_PER_W = _TOTAL_CHUNKS // _NW
_GRP = 16
_NGRP = _PER_W // _GRP
_TROWS = _B * _K * _K

_mesh = plsc.VectorSubcoreMesh(
    core_axis_name="c", subcore_axis_name="s", num_cores=_NC, num_subcores=_NS
)


@functools.partial(
    pl.kernel,
    out_type=(
        jax.ShapeDtypeStruct((_TOTAL_CHUNKS, _CL), jnp.float32),
        jax.ShapeDtypeStruct((_TROWS, 16), jnp.float32),
    ),
    mesh=_mesh,
    compiler_params=pltpu.CompilerParams(needs_layout_passes=False),
    scratch_types=[
        pltpu.VMEM((16,), jnp.int32),
        pltpu.VMEM((16,), jnp.int32),
        pltpu.VMEM((16,), jnp.int32),
        pltpu.VMEM((16,), jnp.int32),
        pltpu.VMEM((_NGRP, 16), jnp.int32),
        pltpu.VMEM((_GRP, _CL), jnp.float32),
        pltpu.VMEM((_GRP, _CL), jnp.float32),
        pltpu.VMEM((_B * _N * _N, 16), jnp.float32),
        pltpu.VMEM((_TROWS, 16), jnp.float32),
        pltpu.SemaphoreType.DMA,
        pltpu.SemaphoreType.DMA,
    ],
)
def _sc_gather(feat_ref, t_ref, keep_ref, rl_ref, out_ref, outt_ref,
               keep_v, rl_v, offs_v, grow_v, idxm, buf0, buf1, t_v,
               tbuf, sem_in, sem_out):
    wid = lax.axis_index("s") * _NC + lax.axis_index("c")
    pltpu.sync_copy(keep_ref, keep_v)
    pltpu.sync_copy(rl_ref, rl_v)

    lane = lax.iota(jnp.int32, 16)
    offs = jnp.zeros((16,), jnp.int32)
    for t in range(_B):
        rl_t = plsc.load_gather(rl_v, [jnp.full((16,), t, jnp.int32)])
        offs = offs + jnp.where(lane > t, rl_t, 0)
    offs_v[...] = offs
    keep = keep_v[...]
    grow_v[...] = plsc.load_gather(offs_v, [lane // _K]) + keep

    base = wid * _PER_W
    for g in range(_NGRP):
        cglob = base + g * _GRP + lane
        row = cglob // _SPLIT
        within = cglob - row * _SPLIT
        idxm[g, :] = plsc.load_gather(grow_v, [row]) * _SPLIT + within

    bufs = (buf0, buf1)
    in_dma = [None, None]
    out_dma = [None, None]
    for g in range(_NGRP):
        p = g % 2
        if out_dma[p] is not None:
            out_dma[p].wait()
        in_dma[p] = pltpu.async_copy(feat_ref.at[idxm.at[g]], bufs[p], sem_in)
        in_dma[p].wait()
        out_dma[p] = pltpu.async_copy(
            bufs[p], out_ref.at[pl.ds(base + g * _GRP, _GRP)], sem_out)
    for p in range(2):
        if out_dma[p] is not None:
            out_dma[p].wait()

    @pl.when(wid == 0)
    def _():
        pltpu.sync_copy(t_ref, t_v)
        srcs = []
        for j3 in range(3):
            jj = j3 * 16 + lane
            b = jj // (_K * _K)
            r3 = jj - b * (_K * _K)
            ki = plsc.load_gather(keep_v, [jnp.minimum(b * _K + r3 // _K, 15)])
            kj = plsc.load_gather(keep_v, [jnp.minimum(b * _K + r3 % _K, 15)])
            src = jnp.minimum(b, _B - 1) * (_N * _N) + ki * _N + kj
            srcs.append(jnp.minimum(src, _B * _N * _N - 1))
        for j in range(_TROWS):
            srcj = jnp.take_along_axis(
                srcs[j // 16], jnp.full((16,), j % 16, jnp.int32), axis=0)
            tbuf[j, :] = plsc.load_gather(t_v, [srcj, lane])
        pltpu.sync_copy(tbuf, outt_ref)


def kernel(feat_map, t_matrix, id_keep, record_len):
    feat2d = feat_map.reshape(_B * _N * _SPLIT, _CL)
    t2d = t_matrix.reshape(_B * _N * _N, 16)
    keep_pad = jnp.zeros((16,), jnp.int32).at[:_B * _K].set(
        id_keep.reshape(-1).astype(jnp.int32))
    rl_pad = jnp.zeros((16,), jnp.int32).at[:_B].set(
        record_len.astype(jnp.int32))
    out_feat, out_t = _sc_gather(feat2d, t2d, keep_pad, rl_pad)
    new_feat_map = out_feat.reshape(_ROWS_OUT, _C, _H, _W)
    new_t_matrix = out_t.reshape(_B, _K, _K, 4, 4)
    new_record_len = jnp.full((_B,), _K, dtype=record_len.dtype)
    return (new_feat_map, new_record_len, new_t_matrix)
